# initial kernel scaffold (unmeasured)
import jax
import jax.numpy as jnp
from jax import lax
from jax.experimental import pallas as pl
from jax.experimental.pallas import tpu as pltpu

N_DEV = 8


def kernel(A, B):
    m_per, k = A.shape
    _, n = B.shape

    a_bf = A.astype(jnp.bfloat16)
    b_bf = B.astype(jnp.bfloat16)

    def body(a_ref, b_ref, out_ref, comm_ref, c_ref, send_sems, recv_sems,
             copy_sem):
        my_pos = lax.axis_index("i")
        left = lax.rem(my_pos + (N_DEV - 1), N_DEV)
        right = lax.rem(my_pos + 1, N_DEV)

        barrier_sem = pltpu.get_barrier_semaphore()
        for nbr in (left, right):
            pl.semaphore_signal(
                barrier_sem, inc=1,
                device_id=(nbr,), device_id_type=pl.DeviceIdType.MESH,
            )
        pl.semaphore_wait(barrier_sem, 2)

        def compute_chunk(origin, a_chunk_ref):
            c_ref[...] = jnp.dot(
                a_chunk_ref[...], b_ref[...],
                preferred_element_type=jnp.float32,
            ).astype(jnp.bfloat16)
            copy = pltpu.make_async_copy(
                c_ref, out_ref.at[pl.ds(origin * m_per, m_per), :], copy_sem
            )
            copy.start()
            copy.wait()

        compute_chunk(my_pos, a_ref)

        for h in range(N_DEV - 1):
            src = a_ref if h == 0 else comm_ref.at[h - 1]
            rdma = pltpu.make_async_remote_copy(
                src_ref=src,
                dst_ref=comm_ref.at[h],
                send_sem=send_sems.at[h],
                recv_sem=recv_sems.at[h],
                device_id=(right,),
                device_id_type=pl.DeviceIdType.MESH,
            )
            rdma.start()
            rdma.wait()
            origin = lax.rem(my_pos + (N_DEV - 1 - h), N_DEV)
            compute_chunk(origin, comm_ref.at[h])

    grid_spec = pltpu.PrefetchScalarGridSpec(
        num_scalar_prefetch=0,
        in_specs=[
            pl.BlockSpec(memory_space=pltpu.VMEM),
            pl.BlockSpec(memory_space=pltpu.VMEM),
        ],
        out_specs=pl.BlockSpec(memory_space=pltpu.ANY),
        scratch_shapes=[
            pltpu.VMEM((N_DEV - 1, m_per, k), jnp.bfloat16),
            pltpu.VMEM((m_per, n), jnp.bfloat16),
            pltpu.SemaphoreType.DMA((N_DEV - 1,)),
            pltpu.SemaphoreType.DMA((N_DEV - 1,)),
            pltpu.SemaphoreType.DMA,
        ],
    )

    return pl.pallas_call(
        body,
        out_shape=jax.ShapeDtypeStruct((N_DEV * m_per, n), jnp.bfloat16),
        grid_spec=grid_spec,
        compiler_params=pltpu.CompilerParams(collective_id=0),
    )(a_bf, b_bf)


# baseline (device time: 489337 ns/iter reference)
import jax
import jax.numpy as jnp
from jax import lax
from jax.experimental import pallas as pl
from jax.experimental.pallas import tpu as pltpu

N_DEV = 8


def kernel(A, B):
    m_per, k = A.shape
    _, n = B.shape

    a_bf = A.astype(jnp.bfloat16)
    b_bf = B.astype(jnp.bfloat16)

    def body(a_ref, b_ref, out_ref, comm_ref, c_ref, send_sems, recv_sems,
             copy_sem):
        my_pos = lax.axis_index("i")
        left = lax.rem(my_pos + (N_DEV - 1), N_DEV)
        right = lax.rem(my_pos + 1, N_DEV)

        barrier_sem = pltpu.get_barrier_semaphore()
        for nbr in (left, right):
            pl.semaphore_signal(
                barrier_sem, inc=1,
                device_id=(nbr,), device_id_type=pl.DeviceIdType.MESH,
            )
        pl.semaphore_wait(barrier_sem, 2)

        def compute_chunk(origin, a_chunk_ref):
            c_ref[...] = jnp.dot(
                a_chunk_ref[...], b_ref[...],
                preferred_element_type=jnp.float32,
            ).astype(jnp.bfloat16)
            copy = pltpu.make_async_copy(
                c_ref, out_ref.at[pl.ds(origin * m_per, m_per), :], copy_sem
            )
            copy.start()
            copy.wait()

        compute_chunk(my_pos, a_ref)

        for h in range(N_DEV - 1):
            src = a_ref if h == 0 else comm_ref.at[h - 1]
            rdma = pltpu.make_async_remote_copy(
                src_ref=src,
                dst_ref=comm_ref.at[h],
                send_sem=send_sems.at[h],
                recv_sem=recv_sems.at[h],
                device_id=(right,),
                device_id_type=pl.DeviceIdType.MESH,
            )
            rdma.start()
            rdma.wait()
            origin = lax.rem(my_pos + (N_DEV - 1 - h), N_DEV)
            compute_chunk(origin, comm_ref.at[h])

    grid_spec = pltpu.PrefetchScalarGridSpec(
        num_scalar_prefetch=0,
        in_specs=[
            pl.BlockSpec(memory_space=pltpu.VMEM),
            pl.BlockSpec(memory_space=pltpu.VMEM),
        ],
        out_specs=pl.BlockSpec(memory_space=pl.ANY),
        scratch_shapes=[
            pltpu.VMEM((N_DEV - 1, m_per, k), jnp.bfloat16),
            pltpu.VMEM((m_per, n), jnp.bfloat16),
            pltpu.SemaphoreType.DMA((N_DEV - 1,)),
            pltpu.SemaphoreType.DMA((N_DEV - 1,)),
            pltpu.SemaphoreType.DMA,
        ],
    )

    return pl.pallas_call(
        body,
        out_shape=jax.ShapeDtypeStruct((N_DEV * m_per, n), jnp.bfloat16),
        grid_spec=grid_spec,
        compiler_params=pltpu.CompilerParams(
            collective_id=0, vmem_limit_bytes=60 * 1024 * 1024
        ),
    )(a_bf, b_bf)


# device time: 246348 ns/iter; 1.9864x vs baseline; 1.9864x over previous
import jax
import jax.numpy as jnp
from jax import lax
from jax.experimental import pallas as pl
from jax.experimental.pallas import tpu as pltpu

N_DEV = 8
N_HOP = N_DEV - 1


def kernel(A, B):
    m_per, k = A.shape
    _, n = B.shape
    half = m_per // 2

    a_bf = A.astype(jnp.bfloat16)
    b_bf = B.astype(jnp.bfloat16)

    def body(a_ref, b_ref, out_ref, cw_ref, ccw_ref, c_cw_ref, c_ccw_ref,
             cw_send, cw_recv, ccw_send, ccw_recv, copy_cw_sem, copy_ccw_sem):
        my_pos = lax.axis_index("i")
        left = lax.rem(my_pos + (N_DEV - 1), N_DEV)
        right = lax.rem(my_pos + 1, N_DEV)

        barrier_sem = pltpu.get_barrier_semaphore()
        for nbr in (left, right):
            pl.semaphore_signal(
                barrier_sem, inc=1,
                device_id=(nbr,), device_id_type=pl.DeviceIdType.MESH,
            )
        pl.semaphore_wait(barrier_sem, 2)

        def mk_cw(h):
            src = a_ref.at[pl.ds(0, half), :] if h == 0 else cw_ref.at[h - 1]
            return pltpu.make_async_remote_copy(
                src_ref=src, dst_ref=cw_ref.at[h],
                send_sem=cw_send.at[h], recv_sem=cw_recv.at[h],
                device_id=(right,), device_id_type=pl.DeviceIdType.MESH,
            )

        def mk_ccw(h):
            src = a_ref.at[pl.ds(half, half), :] if h == 0 else ccw_ref.at[h - 1]
            return pltpu.make_async_remote_copy(
                src_ref=src, dst_ref=ccw_ref.at[h],
                send_sem=ccw_send.at[h], recv_sem=ccw_recv.at[h],
                device_id=(left,), device_id_type=pl.DeviceIdType.MESH,
            )

        def compute_half(origin_row, a_half_ref, c_ref, sem):
            c_ref[...] = jnp.dot(
                a_half_ref[...], b_ref[...],
                preferred_element_type=jnp.float32,
            ).astype(jnp.bfloat16)
            copy = pltpu.make_async_copy(
                c_ref, out_ref.at[pl.ds(origin_row, half), :], sem
            )
            copy.start()
            copy.wait()

        mk_cw(0).start()
        mk_ccw(0).start()

        compute_half(my_pos * m_per, a_ref.at[pl.ds(0, half), :],
                     c_cw_ref, copy_cw_sem)
        compute_half(my_pos * m_per + half, a_ref.at[pl.ds(half, half), :],
                     c_ccw_ref, copy_ccw_sem)

        for h in range(N_HOP):
            mk_cw(h).wait()
            mk_ccw(h).wait()
            if h + 1 < N_HOP:
                mk_cw(h + 1).start()
                mk_ccw(h + 1).start()
            origin_cw = lax.rem(my_pos + (N_DEV - 1 - h), N_DEV)
            origin_ccw = lax.rem(my_pos + (h + 1), N_DEV)
            compute_half(origin_cw * m_per, cw_ref.at[h],
                         c_cw_ref, copy_cw_sem)
            compute_half(origin_ccw * m_per + half, ccw_ref.at[h],
                         c_ccw_ref, copy_ccw_sem)

    grid_spec = pltpu.PrefetchScalarGridSpec(
        num_scalar_prefetch=0,
        in_specs=[
            pl.BlockSpec(memory_space=pltpu.VMEM),
            pl.BlockSpec(memory_space=pltpu.VMEM),
        ],
        out_specs=pl.BlockSpec(memory_space=pl.ANY),
        scratch_shapes=[
            pltpu.VMEM((N_HOP, half, k), jnp.bfloat16),
            pltpu.VMEM((N_HOP, half, k), jnp.bfloat16),
            pltpu.VMEM((half, n), jnp.bfloat16),
            pltpu.VMEM((half, n), jnp.bfloat16),
            pltpu.SemaphoreType.DMA((N_HOP,)),
            pltpu.SemaphoreType.DMA((N_HOP,)),
            pltpu.SemaphoreType.DMA((N_HOP,)),
            pltpu.SemaphoreType.DMA((N_HOP,)),
            pltpu.SemaphoreType.DMA,
            pltpu.SemaphoreType.DMA,
        ],
    )

    return pl.pallas_call(
        body,
        out_shape=jax.ShapeDtypeStruct((N_DEV * m_per, n), jnp.bfloat16),
        grid_spec=grid_spec,
        compiler_params=pltpu.CompilerParams(
            collective_id=0, vmem_limit_bytes=60 * 1024 * 1024
        ),
    )(a_bf, b_bf)


# device time: 181718 ns/iter; 2.6928x vs baseline; 1.3557x over previous
import jax
import jax.numpy as jnp
from jax import lax
from jax.experimental import pallas as pl
from jax.experimental.pallas import tpu as pltpu

N_DEV = 8

MASKS = (1, 3, 4)
ORDERS = ((0, 1, 2), (1, 2, 0), (2, 0, 1))
ROW_OFF = (0, 688, 1376)
ROW_CNT = (688, 688, 672)


def kernel(A, B):
    m_per, k = A.shape
    _, n = B.shape

    a_bf = A.astype(jnp.bfloat16)
    b_bf = B.astype(jnp.bfloat16)

    def body(a_ref, b_ref, out_ref, g0, g1, g2, c0, c1, c2,
             send_sems, recv_sems, cp_sems):
        my = lax.axis_index("i")
        gbufs = (g0, g1, g2)
        cbufs = (c0, c1, c2)
        basis = tuple(tuple(MASKS[d] for d in ORDERS[g]) for g in range(3))

        barrier_sem = pltpu.get_barrier_semaphore()
        for mask in MASKS:
            pl.semaphore_signal(
                barrier_sem, inc=1,
                device_id=(jnp.bitwise_xor(my, mask),),
                device_id_type=pl.DeviceIdType.MESH)
        pl.semaphore_wait(barrier_sem, 3)

        own_cps = []
        for g in range(3):
            cp = pltpu.make_async_copy(
                a_ref.at[pl.ds(ROW_OFF[g], ROW_CNT[g]), :],
                gbufs[g].at[0], cp_sems.at[g])
            cp.start()
            own_cps.append(cp)
        for cp in own_cps:
            cp.wait()

        def offset_of(g, r):
            off = 0
            for bit in range(3):
                if r & (1 << bit):
                    off ^= basis[g][bit]
            return off

        def mk_msg(g, s, mi):
            return pltpu.make_async_remote_copy(
                src_ref=gbufs[g].at[mi],
                dst_ref=gbufs[g].at[(1 << s) + mi],
                send_sem=send_sems.at[g, s, mi],
                recv_sem=recv_sems.at[g, s, mi],
                device_id=(jnp.bitwise_xor(my, basis[g][s]),),
                device_id_type=pl.DeviceIdType.MESH)

        pending = [None, None, None]

        def compute(g, r):
            origin = jnp.bitwise_xor(my, offset_of(g, r))
            if pending[g] is not None:
                pending[g].wait()
            cbufs[g][...] = jnp.dot(
                gbufs[g][r], b_ref[...],
                preferred_element_type=jnp.float32).astype(jnp.bfloat16)
            cp = pltpu.make_async_copy(
                cbufs[g],
                out_ref.at[pl.ds(origin * m_per + ROW_OFF[g], ROW_CNT[g]), :],
                cp_sems.at[g])
            cp.start()
            pending[g] = cp

        for g in range(3):
            mk_msg(g, 0, 0).start()
        for g in range(3):
            compute(g, 0)
        for g in range(3):
            mk_msg(g, 0, 0).wait()
            mk_msg(g, 1, 0).start()
            mk_msg(g, 1, 1).start()
        for g in range(3):
            compute(g, 1)
        for g in range(3):
            mk_msg(g, 1, 0).wait()
            mk_msg(g, 1, 1).wait()
            for mi in range(4):
                mk_msg(g, 2, mi).start()
        for g in range(3):
            compute(g, 2)
            compute(g, 3)
        for mi in range(4):
            for g in range(3):
                mk_msg(g, 2, mi).wait()
            for g in range(3):
                compute(g, 4 + mi)
        for g in range(3):
            pending[g].wait()

    grid_spec = pltpu.PrefetchScalarGridSpec(
        num_scalar_prefetch=0,
        in_specs=[
            pl.BlockSpec(memory_space=pltpu.VMEM),
            pl.BlockSpec(memory_space=pltpu.VMEM),
        ],
        out_specs=pl.BlockSpec(memory_space=pl.ANY),
        scratch_shapes=[
            pltpu.VMEM((N_DEV, ROW_CNT[0], k), jnp.bfloat16),
            pltpu.VMEM((N_DEV, ROW_CNT[1], k), jnp.bfloat16),
            pltpu.VMEM((N_DEV, ROW_CNT[2], k), jnp.bfloat16),
            pltpu.VMEM((ROW_CNT[0], n), jnp.bfloat16),
            pltpu.VMEM((ROW_CNT[1], n), jnp.bfloat16),
            pltpu.VMEM((ROW_CNT[2], n), jnp.bfloat16),
            pltpu.SemaphoreType.DMA((3, 3, 4)),
            pltpu.SemaphoreType.DMA((3, 3, 4)),
            pltpu.SemaphoreType.DMA((3,)),
        ],
    )

    return pl.pallas_call(
        body,
        out_shape=jax.ShapeDtypeStruct((N_DEV * m_per, n), jnp.bfloat16),
        grid_spec=grid_spec,
        compiler_params=pltpu.CompilerParams(
            collective_id=0, vmem_limit_bytes=60 * 1024 * 1024
        ),
    )(a_bf, b_bf)


# device time: 181148 ns/iter; 2.7013x vs baseline; 1.0031x over previous
import jax
import jax.numpy as jnp
from jax import lax
from jax.experimental import pallas as pl
from jax.experimental.pallas import tpu as pltpu

N_DEV = 8

MASKS = (1, 3, 4)
ORDERS = ((0, 1, 2), (1, 2, 0), (2, 0, 1))
ROW_OFF = (0, 688, 1376)
ROW_CNT = (688, 688, 672)


def kernel(A, B):
    m_per, k = A.shape
    _, n = B.shape

    a_bf = A.astype(jnp.bfloat16)
    b_bf = B.astype(jnp.bfloat16)

    def body(a_ref, b_ref, out_ref, g0, g1, g2, c0, c1, c2,
             send_sems, recv_sems, cp_sems):
        my = lax.axis_index("i")
        gbufs = (g0, g1, g2)
        cbufs = (c0, c1, c2)
        basis = tuple(tuple(MASKS[d] for d in ORDERS[g]) for g in range(3))

        barrier_sem = pltpu.get_barrier_semaphore()
        for mask in MASKS:
            pl.semaphore_signal(
                barrier_sem, inc=1,
                device_id=(jnp.bitwise_xor(my, mask),),
                device_id_type=pl.DeviceIdType.MESH)
        pl.semaphore_wait(barrier_sem, 3)

        def own_rows(g):
            return a_ref.at[pl.ds(ROW_OFF[g], ROW_CNT[g]), :]

        def offset_of(g, r):
            off = 0
            for bit in range(3):
                if r & (1 << bit):
                    off ^= basis[g][bit]
            return off

        def mk_msg(g, s, mi):
            return pltpu.make_async_remote_copy(
                src_ref=own_rows(g) if mi == 0 else gbufs[g].at[mi],
                dst_ref=gbufs[g].at[(1 << s) + mi],
                send_sem=send_sems.at[g, s, mi],
                recv_sem=recv_sems.at[g, s, mi],
                device_id=(jnp.bitwise_xor(my, basis[g][s]),),
                device_id_type=pl.DeviceIdType.MESH)

        pending = [None, None, None]

        def compute(g, r):
            origin = jnp.bitwise_xor(my, offset_of(g, r))
            if pending[g] is not None:
                pending[g].wait()
            src = own_rows(g) if r == 0 else gbufs[g].at[r]
            cbufs[g][...] = jnp.dot(
                src[...], b_ref[...],
                preferred_element_type=jnp.float32).astype(jnp.bfloat16)
            cp = pltpu.make_async_copy(
                cbufs[g],
                out_ref.at[pl.ds(origin * m_per + ROW_OFF[g], ROW_CNT[g]), :],
                cp_sems.at[g])
            cp.start()
            pending[g] = cp

        for g in range(3):
            mk_msg(g, 0, 0).start()
        for g in range(3):
            compute(g, 0)
        for g in range(3):
            mk_msg(g, 0, 0).wait()
            mk_msg(g, 1, 0).start()
            mk_msg(g, 1, 1).start()
        for g in range(3):
            compute(g, 1)
        for g in range(3):
            mk_msg(g, 1, 0).wait()
            mk_msg(g, 1, 1).wait()
            for mi in range(4):
                mk_msg(g, 2, mi).start()
        for g in range(3):
            compute(g, 2)
            compute(g, 3)
        for mi in range(4):
            for g in range(3):
                mk_msg(g, 2, mi).wait()
            for g in range(3):
                compute(g, 4 + mi)
        for g in range(3):
            pending[g].wait()

    grid_spec = pltpu.PrefetchScalarGridSpec(
        num_scalar_prefetch=0,
        in_specs=[
            pl.BlockSpec(memory_space=pltpu.VMEM),
            pl.BlockSpec(memory_space=pltpu.VMEM),
        ],
        out_specs=pl.BlockSpec(memory_space=pl.ANY),
        scratch_shapes=[
            pltpu.VMEM((N_DEV, ROW_CNT[0], k), jnp.bfloat16),
            pltpu.VMEM((N_DEV, ROW_CNT[1], k), jnp.bfloat16),
            pltpu.VMEM((N_DEV, ROW_CNT[2], k), jnp.bfloat16),
            pltpu.VMEM((ROW_CNT[0], n), jnp.bfloat16),
            pltpu.VMEM((ROW_CNT[1], n), jnp.bfloat16),
            pltpu.VMEM((ROW_CNT[2], n), jnp.bfloat16),
            pltpu.SemaphoreType.DMA((3, 3, 4)),
            pltpu.SemaphoreType.DMA((3, 3, 4)),
            pltpu.SemaphoreType.DMA((3,)),
        ],
    )

    return pl.pallas_call(
        body,
        out_shape=jax.ShapeDtypeStruct((N_DEV * m_per, n), jnp.bfloat16),
        grid_spec=grid_spec,
        compiler_params=pltpu.CompilerParams(
            collective_id=0, vmem_limit_bytes=60 * 1024 * 1024
        ),
    )(a_bf, b_bf)


# device time: 177218 ns/iter; 2.7612x vs baseline; 1.0222x over previous
import jax
import jax.numpy as jnp
from jax import lax
from jax.experimental import pallas as pl
from jax.experimental.pallas import tpu as pltpu

N_DEV = 8

MASKS = (1, 3, 4)
ORDERS = ((0, 1, 2), (1, 2, 0), (2, 0, 1))
ROW_OFF = (0, 688, 1376)
ROW_CNT = (688, 688, 672)


def kernel(A, B):
    m_per, k = A.shape
    _, n = B.shape

    a_bf = A.astype(jnp.bfloat16)
    b_bf = B.astype(jnp.bfloat16)

    def body(a_ref, b_ref, out_ref, g0, g1, g2, c0, c1, c2,
             send_sems, recv_sems, cp_sems):
        my = lax.axis_index("i")
        gbufs = (g0, g1, g2)
        cbufs = (c0, c1, c2)
        basis = tuple(tuple(MASKS[d] for d in ORDERS[g]) for g in range(3))

        barrier_sem = pltpu.get_barrier_semaphore()
        for mask in MASKS:
            pl.semaphore_signal(
                barrier_sem, inc=1,
                device_id=(jnp.bitwise_xor(my, mask),),
                device_id_type=pl.DeviceIdType.MESH)
        pl.semaphore_wait(barrier_sem, 3)

        def own_rows(g):
            return a_ref.at[pl.ds(ROW_OFF[g], ROW_CNT[g]), :]

        def offset_of(g, r):
            off = 0
            for bit in range(3):
                if r & (1 << bit):
                    off ^= basis[g][bit]
            return off

        def mk_msg(g, s, mi):
            return pltpu.make_async_remote_copy(
                src_ref=own_rows(g) if mi == 0 else gbufs[g].at[mi],
                dst_ref=gbufs[g].at[(1 << s) + mi],
                send_sem=send_sems.at[g, s, mi],
                recv_sem=recv_sems.at[g, s, mi],
                device_id=(jnp.bitwise_xor(my, basis[g][s]),),
                device_id_type=pl.DeviceIdType.MESH)

        def mk_half(g, hi):
            hc = ROW_CNT[g] // 2
            return pltpu.make_async_remote_copy(
                src_ref=gbufs[g].at[3, pl.ds(hi * hc, hc), :],
                dst_ref=gbufs[g].at[7, pl.ds(hi * hc, hc), :],
                send_sem=send_sems.at[g, 2, 3 + hi],
                recv_sem=recv_sems.at[g, 2, 3 + hi],
                device_id=(jnp.bitwise_xor(my, basis[g][2]),),
                device_id_type=pl.DeviceIdType.MESH)

        pending = [None, None, None]

        def compute_rows(g, r, r0, cnt):
            origin = jnp.bitwise_xor(my, offset_of(g, r))
            if pending[g] is not None:
                pending[g].wait()
            src = own_rows(g) if r == 0 else gbufs[g].at[r]
            cbufs[g][pl.ds(0, cnt), :] = jnp.dot(
                src[pl.ds(r0, cnt), :], b_ref[...],
                preferred_element_type=jnp.float32).astype(jnp.bfloat16)
            cp = pltpu.make_async_copy(
                cbufs[g].at[pl.ds(0, cnt), :],
                out_ref.at[
                    pl.ds(origin * m_per + ROW_OFF[g] + r0, cnt), :],
                cp_sems.at[g])
            cp.start()
            pending[g] = cp

        def compute(g, r):
            compute_rows(g, r, 0, ROW_CNT[g])

        for g in range(3):
            mk_msg(g, 0, 0).start()
        for g in range(3):
            compute(g, 0)
        for g in range(3):
            mk_msg(g, 0, 0).wait()
            mk_msg(g, 1, 0).start()
            mk_msg(g, 1, 1).start()
        for g in range(3):
            compute(g, 1)
        for g in range(3):
            mk_msg(g, 1, 0).wait()
            mk_msg(g, 1, 1).wait()
            for mi in range(3):
                mk_msg(g, 2, mi).start()
            mk_half(g, 0).start()
            mk_half(g, 1).start()
        for g in range(3):
            compute(g, 2)
            compute(g, 3)
        for mi in range(3):
            for g in range(3):
                mk_msg(g, 2, mi).wait()
            for g in range(3):
                compute(g, 4 + mi)
        for hi in range(2):
            for g in range(3):
                mk_half(g, hi).wait()
            for g in range(3):
                hc = ROW_CNT[g] // 2
                compute_rows(g, 7, hi * hc, hc)
        for g in range(3):
            pending[g].wait()

    grid_spec = pltpu.PrefetchScalarGridSpec(
        num_scalar_prefetch=0,
        in_specs=[
            pl.BlockSpec(memory_space=pltpu.VMEM),
            pl.BlockSpec(memory_space=pltpu.VMEM),
        ],
        out_specs=pl.BlockSpec(memory_space=pl.ANY),
        scratch_shapes=[
            pltpu.VMEM((N_DEV, ROW_CNT[0], k), jnp.bfloat16),
            pltpu.VMEM((N_DEV, ROW_CNT[1], k), jnp.bfloat16),
            pltpu.VMEM((N_DEV, ROW_CNT[2], k), jnp.bfloat16),
            pltpu.VMEM((ROW_CNT[0], n), jnp.bfloat16),
            pltpu.VMEM((ROW_CNT[1], n), jnp.bfloat16),
            pltpu.VMEM((ROW_CNT[2], n), jnp.bfloat16),
            pltpu.SemaphoreType.DMA((3, 3, 5)),
            pltpu.SemaphoreType.DMA((3, 3, 5)),
            pltpu.SemaphoreType.DMA((3,)),
        ],
    )

    return pl.pallas_call(
        body,
        out_shape=jax.ShapeDtypeStruct((N_DEV * m_per, n), jnp.bfloat16),
        grid_spec=grid_spec,
        compiler_params=pltpu.CompilerParams(
            collective_id=0, vmem_limit_bytes=60 * 1024 * 1024
        ),
    )(a_bf, b_bf)


# device time: 173493 ns/iter; 2.8205x vs baseline; 1.0215x over previous
import jax
import jax.numpy as jnp
from jax import lax
from jax.experimental import pallas as pl
from jax.experimental.pallas import tpu as pltpu

N_DEV = 8

MASKS = (1, 3, 4)
ORDERS = ((0, 1, 2), (1, 2, 0), (2, 0, 1))
ROW_OFF = (0, 688, 1376)
ROW_CNT = (688, 688, 672)


def kernel(A, B):
    m_per, k = A.shape
    _, n = B.shape

    a_bf = A.astype(jnp.bfloat16)
    b_bf = B.astype(jnp.bfloat16)

    def body(a_ref, b_ref, out_ref, g0, g1, g2, c0, c1, c2,
             send_sems, recv_sems, cp_sems):
        my = lax.axis_index("i")
        gbufs = (g0, g1, g2)
        cbufs = (c0, c1, c2)
        basis = tuple(tuple(MASKS[d] for d in ORDERS[g]) for g in range(3))

        barrier_sem = pltpu.get_barrier_semaphore()
        for mask in MASKS:
            pl.semaphore_signal(
                barrier_sem, inc=1,
                device_id=(jnp.bitwise_xor(my, mask),),
                device_id_type=pl.DeviceIdType.MESH)
        pl.semaphore_wait(barrier_sem, 3)

        def own_rows(g):
            return a_ref.at[pl.ds(ROW_OFF[g], ROW_CNT[g]), :]

        def offset_of(g, r):
            off = 0
            for bit in range(3):
                if r & (1 << bit):
                    off ^= basis[g][bit]
            return off

        def mk_msg(g, s, mi):
            return pltpu.make_async_remote_copy(
                src_ref=own_rows(g) if mi == 0 else gbufs[g].at[mi],
                dst_ref=gbufs[g].at[(1 << s) + mi],
                send_sem=send_sems.at[g, s, mi],
                recv_sem=recv_sems.at[g, s, mi],
                device_id=(jnp.bitwise_xor(my, basis[g][s]),),
                device_id_type=pl.DeviceIdType.MESH)

        def mk_half(g, hi):
            hc = ROW_CNT[g] // 2
            return pltpu.make_async_remote_copy(
                src_ref=gbufs[g].at[3, pl.ds(hi * hc, hc), :],
                dst_ref=gbufs[g].at[7, pl.ds(hi * hc, hc), :],
                send_sem=send_sems.at[g, 2, 3 + hi],
                recv_sem=recv_sems.at[g, 2, 3 + hi],
                device_id=(jnp.bitwise_xor(my, basis[g][2]),),
                device_id_type=pl.DeviceIdType.MESH)

        pending = [None, None, None]

        def compute_rows(g, r, r0, cnt):
            origin = jnp.bitwise_xor(my, offset_of(g, r))
            if pending[g] is not None:
                pending[g].wait()
            src = own_rows(g) if r == 0 else gbufs[g].at[r]
            cbufs[g][pl.ds(0, cnt), :] = jnp.dot(
                src[pl.ds(r0, cnt), :], b_ref[...],
                preferred_element_type=jnp.float32).astype(jnp.bfloat16)
            cp = pltpu.make_async_copy(
                cbufs[g].at[pl.ds(0, cnt), :],
                out_ref.at[
                    pl.ds(origin * m_per + ROW_OFF[g] + r0, cnt), :],
                cp_sems.at[g])
            cp.start()
            pending[g] = cp

        def compute(g, r):
            compute_rows(g, r, 0, ROW_CNT[g])

        for g in range(3):
            mk_msg(g, 0, 0).start()
        for g in range(3):
            mk_msg(g, 1, 0).start()
        for g in range(3):
            compute(g, 0)
        for g in range(3):
            mk_msg(g, 0, 0).wait()
        for g in range(3):
            mk_msg(g, 1, 1).start()
        for g in range(3):
            mk_msg(g, 2, 0).start()
            mk_msg(g, 2, 1).start()
        for g in range(3):
            compute(g, 1)
        for g in range(3):
            mk_msg(g, 1, 0).wait()
            mk_msg(g, 1, 1).wait()
        for g in range(3):
            mk_msg(g, 2, 2).start()
            mk_half(g, 0).start()
            mk_half(g, 1).start()
        for g in range(3):
            compute(g, 2)
            compute(g, 3)
        for mi in range(3):
            for g in range(3):
                mk_msg(g, 2, mi).wait()
            for g in range(3):
                compute(g, 4 + mi)
        for hi in range(2):
            for g in range(3):
                mk_half(g, hi).wait()
            for g in range(3):
                hc = ROW_CNT[g] // 2
                compute_rows(g, 7, hi * hc, hc)
        for g in range(3):
            pending[g].wait()

    grid_spec = pltpu.PrefetchScalarGridSpec(
        num_scalar_prefetch=0,
        in_specs=[
            pl.BlockSpec(memory_space=pltpu.VMEM),
            pl.BlockSpec(memory_space=pltpu.VMEM),
        ],
        out_specs=pl.BlockSpec(memory_space=pl.ANY),
        scratch_shapes=[
            pltpu.VMEM((N_DEV, ROW_CNT[0], k), jnp.bfloat16),
            pltpu.VMEM((N_DEV, ROW_CNT[1], k), jnp.bfloat16),
            pltpu.VMEM((N_DEV, ROW_CNT[2], k), jnp.bfloat16),
            pltpu.VMEM((ROW_CNT[0], n), jnp.bfloat16),
            pltpu.VMEM((ROW_CNT[1], n), jnp.bfloat16),
            pltpu.VMEM((ROW_CNT[2], n), jnp.bfloat16),
            pltpu.SemaphoreType.DMA((3, 3, 5)),
            pltpu.SemaphoreType.DMA((3, 3, 5)),
            pltpu.SemaphoreType.DMA((3,)),
        ],
    )

    return pl.pallas_call(
        body,
        out_shape=jax.ShapeDtypeStruct((N_DEV * m_per, n), jnp.bfloat16),
        grid_spec=grid_spec,
        compiler_params=pltpu.CompilerParams(
            collective_id=0, vmem_limit_bytes=60 * 1024 * 1024
        ),
    )(a_bf, b_bf)
